# baseline (speedup 1.0000x reference)
import functools

import jax
import jax.numpy as jnp
from jax import lax
from jax.experimental import pallas as pl
from jax.experimental.pallas import tpu as pltpu
from jax.experimental.pallas import tpu_sc as plsc

B = 4096
VOCAB = 128
EMB = 128
N_DENSE = 13
BM = 1024
NBLK = B // BM
AM = 512
ANBLK = B // AM

_NC = 2
_NS = 16
_NW = _NC * _NS
_L = 16


def _sc_fm_dot(emb, idx, fwe):
    bpw = B // _NW

    mesh = plsc.VectorSubcoreMesh(core_axis_name="c", subcore_axis_name="s")

    @functools.partial(
        pl.kernel,
        mesh=mesh,
        out_type=jax.ShapeDtypeStruct((B, _L), jnp.float32),
        scratch_types=[
            pltpu.VMEM((VOCAB, EMB), jnp.float32),
            pltpu.VMEM((bpw + _L,), jnp.int32),
            pltpu.VMEM((EMB,), jnp.float32),
            pltpu.VMEM((bpw, _L), jnp.float32),
        ],
    )
    def gk(emb_hbm, idx_hbm, fwe_hbm, out_hbm, table_v, idx_v, fwe_v, bp_v):
        wid = lax.axis_index("s") * _NC + lax.axis_index("c")
        base = wid * bpw
        pltpu.sync_copy(emb_hbm, table_v)
        pltpu.sync_copy(idx_hbm.at[pl.ds(base, bpw)], idx_v.at[pl.ds(0, bpw)])
        pltpu.sync_copy(fwe_hbm, fwe_v)

        fw_chunks = [fwe_v[pl.ds(c * _L, _L)] for c in range(EMB // _L)]

        def body(r, _):
            v = idx_v[pl.ds(r, _L)][0]
            acc = table_v[v, pl.ds(0, _L)] * fw_chunks[0]
            for c in range(1, EMB // _L):
                acc = acc + table_v[v, pl.ds(c * _L, _L)] * fw_chunks[c]
            bp_v[r, :] = acc
            return 0

        lax.fori_loop(0, bpw, body, 0)
        pltpu.sync_copy(bp_v, out_hbm.at[pl.ds(base, bpw)])

    return gk(emb, idx, fwe)


def _mlp_body(idxr_ref, emb_ref, dnt_ref, w0t_ref, b0_ref, w1t_ref,
              b1_ref, w2t_ref, b2_ref, wo_ref, bo_ref, a_ref):
    bf = jnp.bfloat16
    nt = (((1,), (0,)), ((), ()))
    tt = (((0,), (0,)), ((), ()))
    onehot_t = (lax.broadcasted_iota(jnp.int32, (VOCAB, 1), 0) ==
                idxr_ref[...]).astype(bf)
    e = lax.dot_general(onehot_t, emb_ref[...].astype(bf), tt,
                        preferred_element_type=jnp.float32).astype(bf)
    w0t = w0t_ref[...].astype(bf)
    h = lax.dot_general(e, w0t[:EMB, :], nt, preferred_element_type=jnp.float32)
    h += lax.dot_general(dnt_ref[...].astype(bf), w0t[EMB:, :], tt,
                         preferred_element_type=jnp.float32)
    h = jnp.maximum(h + b0_ref[...], 0.0).astype(bf)
    h = lax.dot_general(h, w1t_ref[...].astype(bf), nt,
                        preferred_element_type=jnp.float32)
    h = jnp.maximum(h + b1_ref[...], 0.0).astype(bf)
    h = lax.dot_general(h, w2t_ref[...].astype(bf), nt,
                        preferred_element_type=jnp.float32)
    h = jnp.maximum(h + b2_ref[...], 0.0)
    a_ref[...] = jnp.sum(h * wo_ref[...], axis=1, keepdims=True) + bo_ref[0, 0]


def _bcast_body(a_ref, bp_ref, dnt_ref, fwd_ref, fb_ref, out_ref, brow_s):
    i = pl.program_id(0)

    @pl.when(i == 0)
    def _():
        brow = lax.dot_general(jnp.ones((1, _L), jnp.float32), bp_ref[...],
                               (((1,), (1,)), ((), ())),
                               preferred_element_type=jnp.float32)
        brow += lax.dot_general(fwd_ref[...], dnt_ref[...],
                                (((1,), (0,)), ((), ())),
                                preferred_element_type=jnp.float32)
        brow_s[...] = brow + fb_ref[0, 0]

    a = a_ref[pl.ds(i * BM, BM), :]
    out_ref[...] = 0.5 * jnp.tanh(0.5 * (a + brow_s[...])) + 0.5


def _full(shape):
    return pl.BlockSpec(shape, lambda i: (0,) * len(shape))


def _mlp_call(idxr, emb, dnt, w0t, b0, w1t, b1, w2t, b2, wo, bo):
    return pl.pallas_call(
        _mlp_body,
        grid=(ANBLK,),
        in_specs=[
            pl.BlockSpec((1, AM), lambda i: (0, i)),
            _full((VOCAB, EMB)),
            pl.BlockSpec((N_DENSE, AM), lambda i: (0, i)),
            _full(w0t.shape),
            _full(b0.shape),
            _full(w1t.shape),
            _full(b1.shape),
            _full(w2t.shape),
            _full(b2.shape),
            _full(wo.shape),
            _full(bo.shape),
        ],
        out_specs=pl.BlockSpec((AM, 1), lambda i: (i, 0)),
        out_shape=jax.ShapeDtypeStruct((B, 1), jnp.float32),
    )(idxr, emb, dnt, w0t, b0, w1t, b1, w2t, b2, wo, bo)


def _bcast_call(a, bp, dnt, fwd, fb):
    args = (a, bp, dnt, fwd, fb)
    return pl.pallas_call(
        _bcast_body,
        grid=(NBLK,),
        in_specs=[_full(x.shape) for x in args],
        out_specs=pl.BlockSpec((BM, B), lambda i: (i, 0)),
        out_shape=jax.ShapeDtypeStruct((B, B), jnp.float32),
        scratch_shapes=[pltpu.VMEM((1, B), jnp.float32)],
    )(*args)


def kernel(sparse_inputs, dense_inputs, emb, fm_w, fm_b, w0, b0, w1, b1, w2,
           b2, wo, bo):
    idx32 = sparse_inputs.astype(jnp.int32)
    idx = idx32.reshape(-1)
    idxr = idx32.reshape(1, B)
    dnt = dense_inputs.T
    fwe = fm_w[:, :EMB]
    fwd = fm_w[:, EMB:]

    bp = _sc_fm_dot(emb, idx, fwe.reshape(-1))

    a = _mlp_call(
        idxr, emb, dnt, w0.T, b0.reshape(1, -1), w1.T, b1.reshape(1, -1),
        w2.T, b2.reshape(1, -1), wo, bo.reshape(1, 1))

    return _bcast_call(a, bp, dnt, fwd, fm_b.reshape(1, 1))


# speedup vs baseline: 1.0197x; 1.0197x over previous
import functools

import jax
import jax.numpy as jnp
from jax import lax
from jax.experimental import pallas as pl
from jax.experimental.pallas import tpu as pltpu
from jax.experimental.pallas import tpu_sc as plsc

B = 4096
VOCAB = 128
EMB = 128
N_DENSE = 13
BM = 512
NBLK = B // BM
AM = 512
ANBLK = B // AM

_NC = 2
_NS = 16
_NW = _NC * _NS
_L = 16


def _sc_fm_dot(emb, idx, fwe):
    bpw = B // _NW

    mesh = plsc.VectorSubcoreMesh(core_axis_name="c", subcore_axis_name="s")

    @functools.partial(
        pl.kernel,
        mesh=mesh,
        out_type=jax.ShapeDtypeStruct((B, _L), jnp.float32),
        scratch_types=[
            pltpu.VMEM((VOCAB, EMB), jnp.float32),
            pltpu.VMEM((bpw + _L,), jnp.int32),
            pltpu.VMEM((EMB,), jnp.float32),
            pltpu.VMEM((bpw, _L), jnp.float32),
        ],
    )
    def gk(emb_hbm, idx_hbm, fwe_hbm, out_hbm, table_v, idx_v, fwe_v, bp_v):
        wid = lax.axis_index("s") * _NC + lax.axis_index("c")
        base = wid * bpw
        pltpu.sync_copy(emb_hbm, table_v)
        pltpu.sync_copy(idx_hbm.at[pl.ds(base, bpw)], idx_v.at[pl.ds(0, bpw)])
        pltpu.sync_copy(fwe_hbm, fwe_v)

        fw_chunks = [fwe_v[pl.ds(c * _L, _L)] for c in range(EMB // _L)]

        def body(r, _):
            v = idx_v[pl.ds(r, _L)][0]
            acc = table_v[v, pl.ds(0, _L)] * fw_chunks[0]
            for c in range(1, EMB // _L):
                acc = acc + table_v[v, pl.ds(c * _L, _L)] * fw_chunks[c]
            bp_v[r, :] = acc
            return 0

        lax.fori_loop(0, bpw, body, 0)
        pltpu.sync_copy(bp_v, out_hbm.at[pl.ds(base, bpw)])

    return gk(emb, idx, fwe)


def _mlp_body(idxr_ref, emb_ref, dnt_ref, w0t_ref, b0_ref, w1t_ref,
              b1_ref, w2t_ref, b2_ref, wo_ref, bo_ref, a_ref):
    bf = jnp.bfloat16
    nt = (((1,), (0,)), ((), ()))
    tt = (((0,), (0,)), ((), ()))
    onehot_t = (lax.broadcasted_iota(jnp.int32, (VOCAB, 1), 0) ==
                idxr_ref[...]).astype(bf)
    e = lax.dot_general(onehot_t, emb_ref[...].astype(bf), tt,
                        preferred_element_type=jnp.float32).astype(bf)
    w0t = w0t_ref[...].astype(bf)
    h = lax.dot_general(e, w0t[:EMB, :], nt, preferred_element_type=jnp.float32)
    h += lax.dot_general(dnt_ref[...].astype(bf), w0t[EMB:, :], tt,
                         preferred_element_type=jnp.float32)
    h = jnp.maximum(h + b0_ref[...], 0.0).astype(bf)
    h = lax.dot_general(h, w1t_ref[...].astype(bf), nt,
                        preferred_element_type=jnp.float32)
    h = jnp.maximum(h + b1_ref[...], 0.0).astype(bf)
    h = lax.dot_general(h, w2t_ref[...].astype(bf), nt,
                        preferred_element_type=jnp.float32)
    h = jnp.maximum(h + b2_ref[...], 0.0)
    a_ref[...] = jnp.sum(h * wo_ref[...], axis=1, keepdims=True) + bo_ref[0, 0]


def _bcast_body(a_ref, bp_ref, dnt_ref, fwd_ref, fb_ref, out_ref, brow_s):
    i = pl.program_id(0)

    @pl.when(i == 0)
    def _():
        brow = lax.dot_general(jnp.ones((1, _L), jnp.float32), bp_ref[...],
                               (((1,), (1,)), ((), ())),
                               preferred_element_type=jnp.float32)
        brow += lax.dot_general(fwd_ref[...], dnt_ref[...],
                                (((1,), (0,)), ((), ())),
                                preferred_element_type=jnp.float32)
        brow_s[...] = brow + fb_ref[0, 0]

    a = a_ref[pl.ds(i * BM, BM), :]
    out_ref[...] = 0.5 * jnp.tanh(0.5 * (a + brow_s[...])) + 0.5


def _full(shape):
    return pl.BlockSpec(shape, lambda i: (0,) * len(shape))


def _mlp_call(idxr, emb, dnt, w0t, b0, w1t, b1, w2t, b2, wo, bo):
    return pl.pallas_call(
        _mlp_body,
        grid=(ANBLK,),
        in_specs=[
            pl.BlockSpec((1, AM), lambda i: (0, i)),
            _full((VOCAB, EMB)),
            pl.BlockSpec((N_DENSE, AM), lambda i: (0, i)),
            _full(w0t.shape),
            _full(b0.shape),
            _full(w1t.shape),
            _full(b1.shape),
            _full(w2t.shape),
            _full(b2.shape),
            _full(wo.shape),
            _full(bo.shape),
        ],
        out_specs=pl.BlockSpec((AM, 1), lambda i: (i, 0)),
        out_shape=jax.ShapeDtypeStruct((B, 1), jnp.float32),
    )(idxr, emb, dnt, w0t, b0, w1t, b1, w2t, b2, wo, bo)


def _bcast_call(a, bp, dnt, fwd, fb):
    args = (a, bp, dnt, fwd, fb)
    return pl.pallas_call(
        _bcast_body,
        grid=(NBLK,),
        in_specs=[_full(x.shape) for x in args],
        out_specs=pl.BlockSpec((BM, B), lambda i: (i, 0)),
        out_shape=jax.ShapeDtypeStruct((B, B), jnp.float32),
        scratch_shapes=[pltpu.VMEM((1, B), jnp.float32)],
    )(*args)


def kernel(sparse_inputs, dense_inputs, emb, fm_w, fm_b, w0, b0, w1, b1, w2,
           b2, wo, bo):
    idx32 = sparse_inputs.astype(jnp.int32)
    idx = idx32.reshape(-1)
    idxr = idx32.reshape(1, B)
    dnt = dense_inputs.T
    fwe = fm_w[:, :EMB]
    fwd = fm_w[:, EMB:]

    bp = _sc_fm_dot(emb, idx, fwe.reshape(-1))

    a = _mlp_call(
        idxr, emb, dnt, w0.T, b0.reshape(1, -1), w1.T, b1.reshape(1, -1),
        w2.T, b2.reshape(1, -1), wo, bo.reshape(1, 1))

    return _bcast_call(a, bp, dnt, fwd, fm_b.reshape(1, 1))


# speedup vs baseline: 1.0564x; 1.0360x over previous
import functools

import jax
import jax.numpy as jnp
from jax import lax
from jax.experimental import pallas as pl
from jax.experimental.pallas import tpu as pltpu
from jax.experimental.pallas import tpu_sc as plsc

B = 4096
VOCAB = 128
EMB = 128
N_DENSE = 13
BM = 512
NBLK = B // BM
AM = 1024
ANBLK = B // AM

_NC = 2
_NS = 16
_NW = _NC * _NS
_L = 16


def _sc_fm_dot(emb, idx, fwe):
    bpw = B // _NW

    mesh = plsc.VectorSubcoreMesh(core_axis_name="c", subcore_axis_name="s")

    @functools.partial(
        pl.kernel,
        mesh=mesh,
        out_type=jax.ShapeDtypeStruct((B, _L), jnp.float32),
        scratch_types=[
            pltpu.VMEM((VOCAB, EMB), jnp.float32),
            pltpu.VMEM((bpw + _L,), jnp.int32),
            pltpu.VMEM((EMB,), jnp.float32),
            pltpu.VMEM((bpw, _L), jnp.float32),
        ],
    )
    def gk(emb_hbm, idx_hbm, fwe_hbm, out_hbm, table_v, idx_v, fwe_v, bp_v):
        wid = lax.axis_index("s") * _NC + lax.axis_index("c")
        base = wid * bpw
        pltpu.sync_copy(emb_hbm, table_v)
        pltpu.sync_copy(idx_hbm.at[pl.ds(base, bpw)], idx_v.at[pl.ds(0, bpw)])
        pltpu.sync_copy(fwe_hbm, fwe_v)

        fw_chunks = [fwe_v[pl.ds(c * _L, _L)] for c in range(EMB // _L)]

        def body(r, _):
            v = idx_v[pl.ds(r, _L)][0]
            acc = table_v[v, pl.ds(0, _L)] * fw_chunks[0]
            for c in range(1, EMB // _L):
                acc = acc + table_v[v, pl.ds(c * _L, _L)] * fw_chunks[c]
            bp_v[r, :] = acc
            return 0

        lax.fori_loop(0, bpw, body, 0)
        pltpu.sync_copy(bp_v, out_hbm.at[pl.ds(base, bpw)])

    return gk(emb, idx, fwe)


def _mlp_body(idxr_ref, emb_ref, dnt_ref, w0t_ref, b0_ref, w1t_ref,
              b1_ref, w2t_ref, b2_ref, wo_ref, bo_ref, a_ref):
    bf = jnp.bfloat16
    nt = (((1,), (0,)), ((), ()))
    tt = (((0,), (0,)), ((), ()))
    onehot_t = (lax.broadcasted_iota(jnp.int32, (VOCAB, 1), 0) ==
                idxr_ref[...]).astype(bf)
    e = lax.dot_general(onehot_t, emb_ref[...].astype(bf), tt,
                        preferred_element_type=jnp.float32).astype(bf)
    w0t = w0t_ref[...].astype(bf)
    h = lax.dot_general(e, w0t[:EMB, :], nt, preferred_element_type=jnp.float32)
    h += lax.dot_general(dnt_ref[...].astype(bf), w0t[EMB:, :], tt,
                         preferred_element_type=jnp.float32)
    h = jnp.maximum(h + b0_ref[...], 0.0).astype(bf)
    h = lax.dot_general(h, w1t_ref[...].astype(bf), nt,
                        preferred_element_type=jnp.float32)
    h = jnp.maximum(h + b1_ref[...], 0.0).astype(bf)
    h = lax.dot_general(h, w2t_ref[...].astype(bf), nt,
                        preferred_element_type=jnp.float32)
    h = jnp.maximum(h + b2_ref[...], 0.0)
    a_ref[...] = jnp.sum(h * wo_ref[...], axis=1, keepdims=True) + bo_ref[0, 0]


def _bcast_body(a_ref, bp_ref, dnt_ref, fwd_ref, fb_ref, out_ref, brow_s):
    i = pl.program_id(0)

    @pl.when(i == 0)
    def _():
        brow = lax.dot_general(jnp.ones((1, _L), jnp.float32), bp_ref[...],
                               (((1,), (1,)), ((), ())),
                               preferred_element_type=jnp.float32)
        brow += lax.dot_general(fwd_ref[...], dnt_ref[...],
                                (((1,), (0,)), ((), ())),
                                preferred_element_type=jnp.float32)
        brow_s[...] = brow + fb_ref[0, 0]

    a = a_ref[pl.ds(i * BM, BM), :]
    out_ref[...] = 0.5 * jnp.tanh(0.5 * (a + brow_s[...])) + 0.5


def _full(shape):
    return pl.BlockSpec(shape, lambda i: (0,) * len(shape))


def _mlp_call(idxr, emb, dnt, w0t, b0, w1t, b1, w2t, b2, wo, bo):
    return pl.pallas_call(
        _mlp_body,
        grid=(ANBLK,),
        in_specs=[
            pl.BlockSpec((1, AM), lambda i: (0, i)),
            _full((VOCAB, EMB)),
            pl.BlockSpec((N_DENSE, AM), lambda i: (0, i)),
            _full(w0t.shape),
            _full(b0.shape),
            _full(w1t.shape),
            _full(b1.shape),
            _full(w2t.shape),
            _full(b2.shape),
            _full(wo.shape),
            _full(bo.shape),
        ],
        out_specs=pl.BlockSpec((AM, 1), lambda i: (i, 0)),
        out_shape=jax.ShapeDtypeStruct((B, 1), jnp.float32),
    )(idxr, emb, dnt, w0t, b0, w1t, b1, w2t, b2, wo, bo)


def _bcast_call(a, bp, dnt, fwd, fb):
    args = (a, bp, dnt, fwd, fb)
    return pl.pallas_call(
        _bcast_body,
        grid=(NBLK,),
        in_specs=[_full(x.shape) for x in args],
        out_specs=pl.BlockSpec((BM, B), lambda i: (i, 0)),
        out_shape=jax.ShapeDtypeStruct((B, B), jnp.float32),
        scratch_shapes=[pltpu.VMEM((1, B), jnp.float32)],
    )(*args)


def kernel(sparse_inputs, dense_inputs, emb, fm_w, fm_b, w0, b0, w1, b1, w2,
           b2, wo, bo):
    idx32 = sparse_inputs.astype(jnp.int32)
    idx = idx32.reshape(-1)
    idxr = idx32.reshape(1, B)
    dnt = dense_inputs.T
    fwe = fm_w[:, :EMB]
    fwd = fm_w[:, EMB:]

    bp = _sc_fm_dot(emb, idx, fwe.reshape(-1))

    a = _mlp_call(
        idxr, emb, dnt, w0.T, b0.reshape(1, -1), w1.T, b1.reshape(1, -1),
        w2.T, b2.reshape(1, -1), wo, bo.reshape(1, 1))

    return _bcast_call(a, bp, dnt, fwd, fm_b.reshape(1, 1))


# speedup vs baseline: 1.0571x; 1.0006x over previous
import functools

import jax
import jax.numpy as jnp
from jax import lax
from jax.experimental import pallas as pl
from jax.experimental.pallas import tpu as pltpu
from jax.experimental.pallas import tpu_sc as plsc

B = 4096
VOCAB = 128
EMB = 128
N_DENSE = 13
BM = 512
NBLK = B // BM
AM = 1024
ANBLK = B // AM

_NC = 1
_NS = 16
_NW = _NC * _NS
_L = 16


def _sc_fm_dot(emb, idx, fwe):
    bpw = B // _NW

    mesh = plsc.VectorSubcoreMesh(core_axis_name="c", subcore_axis_name="s", num_cores=1)

    @functools.partial(
        pl.kernel,
        mesh=mesh,
        out_type=jax.ShapeDtypeStruct((B, _L), jnp.float32),
        scratch_types=[
            pltpu.VMEM((VOCAB, EMB), jnp.float32),
            pltpu.VMEM((bpw + _L,), jnp.int32),
            pltpu.VMEM((EMB,), jnp.float32),
            pltpu.VMEM((bpw, _L), jnp.float32),
        ],
    )
    def gk(emb_hbm, idx_hbm, fwe_hbm, out_hbm, table_v, idx_v, fwe_v, bp_v):
        wid = lax.axis_index("s") * _NC + lax.axis_index("c")
        base = wid * bpw
        pltpu.sync_copy(emb_hbm, table_v)
        pltpu.sync_copy(idx_hbm.at[pl.ds(base, bpw)], idx_v.at[pl.ds(0, bpw)])
        pltpu.sync_copy(fwe_hbm, fwe_v)

        fw_chunks = [fwe_v[pl.ds(c * _L, _L)] for c in range(EMB // _L)]

        def body(r, _):
            v = idx_v[pl.ds(r, _L)][0]
            acc = table_v[v, pl.ds(0, _L)] * fw_chunks[0]
            for c in range(1, EMB // _L):
                acc = acc + table_v[v, pl.ds(c * _L, _L)] * fw_chunks[c]
            bp_v[r, :] = acc
            return 0

        lax.fori_loop(0, bpw, body, 0)
        pltpu.sync_copy(bp_v, out_hbm.at[pl.ds(base, bpw)])

    return gk(emb, idx, fwe)


def _mlp_body(idxr_ref, emb_ref, dnt_ref, w0t_ref, b0_ref, w1t_ref,
              b1_ref, w2t_ref, b2_ref, wo_ref, bo_ref, a_ref):
    bf = jnp.bfloat16
    nt = (((1,), (0,)), ((), ()))
    tt = (((0,), (0,)), ((), ()))
    onehot_t = (lax.broadcasted_iota(jnp.int32, (VOCAB, 1), 0) ==
                idxr_ref[...]).astype(bf)
    e = lax.dot_general(onehot_t, emb_ref[...].astype(bf), tt,
                        preferred_element_type=jnp.float32).astype(bf)
    w0t = w0t_ref[...].astype(bf)
    h = lax.dot_general(e, w0t[:EMB, :], nt, preferred_element_type=jnp.float32)
    h += lax.dot_general(dnt_ref[...].astype(bf), w0t[EMB:, :], tt,
                         preferred_element_type=jnp.float32)
    h = jnp.maximum(h + b0_ref[...], 0.0).astype(bf)
    h = lax.dot_general(h, w1t_ref[...].astype(bf), nt,
                        preferred_element_type=jnp.float32)
    h = jnp.maximum(h + b1_ref[...], 0.0).astype(bf)
    h = lax.dot_general(h, w2t_ref[...].astype(bf), nt,
                        preferred_element_type=jnp.float32)
    h = jnp.maximum(h + b2_ref[...], 0.0)
    a_ref[...] = jnp.sum(h * wo_ref[...], axis=1, keepdims=True) + bo_ref[0, 0]


def _bcast_body(a_ref, bp_ref, dnt_ref, fwd_ref, fb_ref, out_ref, brow_s):
    i = pl.program_id(0)

    @pl.when(i == 0)
    def _():
        brow = lax.dot_general(jnp.ones((1, _L), jnp.float32), bp_ref[...],
                               (((1,), (1,)), ((), ())),
                               preferred_element_type=jnp.float32)
        brow += lax.dot_general(fwd_ref[...], dnt_ref[...],
                                (((1,), (0,)), ((), ())),
                                preferred_element_type=jnp.float32)
        brow_s[...] = brow + fb_ref[0, 0]

    a = a_ref[pl.ds(i * BM, BM), :]
    out_ref[...] = 0.5 * jnp.tanh(0.5 * (a + brow_s[...])) + 0.5


def _full(shape):
    return pl.BlockSpec(shape, lambda i: (0,) * len(shape))


def _mlp_call(idxr, emb, dnt, w0t, b0, w1t, b1, w2t, b2, wo, bo):
    return pl.pallas_call(
        _mlp_body,
        grid=(ANBLK,),
        in_specs=[
            pl.BlockSpec((1, AM), lambda i: (0, i)),
            _full((VOCAB, EMB)),
            pl.BlockSpec((N_DENSE, AM), lambda i: (0, i)),
            _full(w0t.shape),
            _full(b0.shape),
            _full(w1t.shape),
            _full(b1.shape),
            _full(w2t.shape),
            _full(b2.shape),
            _full(wo.shape),
            _full(bo.shape),
        ],
        out_specs=pl.BlockSpec((AM, 1), lambda i: (i, 0)),
        out_shape=jax.ShapeDtypeStruct((B, 1), jnp.float32),
    )(idxr, emb, dnt, w0t, b0, w1t, b1, w2t, b2, wo, bo)


def _bcast_call(a, bp, dnt, fwd, fb):
    args = (a, bp, dnt, fwd, fb)
    return pl.pallas_call(
        _bcast_body,
        grid=(NBLK,),
        in_specs=[_full(x.shape) for x in args],
        out_specs=pl.BlockSpec((BM, B), lambda i: (i, 0)),
        out_shape=jax.ShapeDtypeStruct((B, B), jnp.float32),
        scratch_shapes=[pltpu.VMEM((1, B), jnp.float32)],
    )(*args)


def kernel(sparse_inputs, dense_inputs, emb, fm_w, fm_b, w0, b0, w1, b1, w2,
           b2, wo, bo):
    idx32 = sparse_inputs.astype(jnp.int32)
    idx = idx32.reshape(-1)
    idxr = idx32.reshape(1, B)
    dnt = dense_inputs.T
    fwe = fm_w[:, :EMB]
    fwd = fm_w[:, EMB:]

    bp = _sc_fm_dot(emb, idx, fwe.reshape(-1))

    a = _mlp_call(
        idxr, emb, dnt, w0.T, b0.reshape(1, -1), w1.T, b1.reshape(1, -1),
        w2.T, b2.reshape(1, -1), wo, bo.reshape(1, 1))

    return _bcast_call(a, bp, dnt, fwd, fm_b.reshape(1, 1))
